# baseline (speedup 1.0000x reference)
import jax
import jax.numpy as jnp
from jax import lax
from jax.experimental import pallas as pl
from jax.experimental.pallas import tpu as pltpu
from jax.experimental.pallas import tpu_sc as plsc

_DIM = 32
_B = 16384
_L = 50
_NC = 2
_NS = 16
_NW = _NC * _NS
_BPW = _B // _NW
_NV = _BPW // 16


def _body(x_hbm, m_hbm, w_hbm, out_hbm, xb, mb, idx, acc, sem):
    c = lax.axis_index("c")
    s = lax.axis_index("s")
    wid = s * _NC + c
    base = wid * _BPW

    def load_select(l):
        pltpu.sync_copy(x_hbm.at[l, pl.ds(base, _BPW)], xb)
        pltpu.sync_copy(m_hbm.at[l, pl.ds(base, _BPW)], mb)

        def sel(i, carry):
            o = pl.multiple_of(i * 16, 16)
            xv = xb[pl.ds(o, 16)]
            mv = mb[pl.ds(o, 16)]
            idx[pl.ds(o, 16)] = jnp.where(mv > 0, xv, 0)
            return carry

        lax.fori_loop(0, _NV, sel, 0)

    load_select(0)
    pltpu.async_copy(w_hbm.at[idx], acc, sem).wait()

    def step(l, carry):
        load_select(l)
        pltpu.async_copy(w_hbm.at[idx], acc, sem, add=True).wait()
        return carry

    lax.fori_loop(1, _L, step, 0)

    pltpu.sync_copy(acc, out_hbm.at[pl.ds(base, _BPW)])


def kernel(x, mask, W):
    xt = x.T
    mt = mask[:, :, 0].astype(jnp.int32).T
    mesh = plsc.VectorSubcoreMesh(
        core_axis_name="c", subcore_axis_name="s",
        num_cores=_NC, num_subcores=_NS,
    )
    k = pl.kernel(
        _body,
        out_type=jax.ShapeDtypeStruct((_B, _DIM), jnp.float32),
        mesh=mesh,
        compiler_params=pltpu.CompilerParams(use_tc_tiling_on_sc=False),
        scratch_types=[
            pltpu.VMEM((_BPW,), jnp.int32),
            pltpu.VMEM((_BPW,), jnp.int32),
            pltpu.VMEM((_BPW,), jnp.int32),
            pltpu.VMEM((_BPW, _DIM), jnp.float32),
            pltpu.SemaphoreType.DMA,
        ],
    )
    return k(xt, mt, W)
